# baseline (speedup 1.0000x reference)
import math

import jax
import jax.numpy as jnp
from jax import lax
from jax.experimental import pallas as pl
from jax.experimental.pallas import tpu as pltpu
from jax.experimental.pallas import tpu_sc as plsc

_NUM_LEVELS = 16
_BASE_RES = 16
_DESIRED_RES = 2048
_LOG2_HASH = 19
_PRIME_I32 = 2654435761 - (1 << 32)
_HASH_MASK = (1 << _LOG2_HASH) - 1

_B = 262144
_NC, _NS = 2, 16
_NW = _NC * _NS
_CHUNK = 512
_PTS_PER_W = _B // _NW
_NCHUNKS = _PTS_PER_W // _CHUNK
_NGROUPS = _CHUNK // 16
_PLANES = ((0, 1), (0, 2), (1, 2))
_NIDX = 3 * 4 * _CHUNK


def _levels():
    L = _NUM_LEVELS
    b = math.exp(math.log(_DESIRED_RES / _BASE_RES) / (L - 1))
    H = 1 << _LOG2_HASH
    out, off = [], 0
    for l in range(L):
        res = int(math.ceil(_BASE_RES * (b ** l)))
        hashed = (res + 1) ** 2 > H
        size = H if hashed else (res + 1) ** 2
        out.append((res, hashed, off))
        off += size
    return out, off


_LEVEL_PARAMS, _ROWS_PER_PLANE = _levels()


def _phase_a(level, xbuf, fracb, idxe, idxo):
    res, hashed, loff = _LEVEL_PARAMS[level]
    resf = float(res)
    stride = res + 1

    @pl.loop(0, _NGROUPS)
    def _g(i):
        s16 = i * 16
        ip0, ip1 = [], []
        for c in range(3):
            xv = xbuf[c, pl.ds(s16, 16)]
            pos = xv * resf
            ip = pos.astype(jnp.int32)
            fracb[c, pl.ds(s16, 16)] = pos - ip.astype(jnp.float32)
            ip0.append(ip)
            ip1.append(jnp.minimum(ip + 1, res))
        for p, (ca, cb) in enumerate(_PLANES):
            off = p * _ROWS_PER_PLANE + loff
            a0, a1 = ip0[ca], ip1[ca]
            if not hashed:
                b0 = ip0[cb] * stride + off
                b1 = ip1[cb] * stride + off
                corners = (a0 + b0, a0 + b1, a1 + b0, a1 + b1)
            else:
                h0 = ip0[cb] * _PRIME_I32
                h1 = ip1[cb] * _PRIME_I32
                corners = (((a0 ^ h0) & _HASH_MASK) + off,
                           ((a0 ^ h1) & _HASH_MASK) + off,
                           ((a1 ^ h0) & _HASH_MASK) + off,
                           ((a1 ^ h1) & _HASH_MASK) + off)
            for c4, v in enumerate(corners):
                e = v + v
                idxe[pl.ds(p * 4 * _CHUNK + c4 * _CHUNK + s16, 16)] = e
                idxo[pl.ds(p * 4 * _CHUNK + c4 * _CHUNK + s16, 16)] = e + 1


def _phase_b(level, fracb, rows0, rows1, acc):
    @pl.loop(0, _NGROUPS)
    def _g(i):
        s16 = i * 16
        for p, (ca, cb) in enumerate(_PLANES):
            fx = fracb[ca, pl.ds(s16, 16)]
            fy = fracb[cb, pl.ds(s16, 16)]
            for d, rows in enumerate((rows0, rows1)):
                base = p * 4 * _CHUNK + s16
                g00 = rows[pl.ds(base, 16)]
                g01 = rows[pl.ds(base + _CHUNK, 16)]
                g10 = rows[pl.ds(base + 2 * _CHUNK, 16)]
                g11 = rows[pl.ds(base + 3 * _CHUNK, 16)]
                v0 = g00 + fy * (g01 - g00)
                v1 = g10 + fy * (g11 - g10)
                v = v0 + fx * (v1 - v0)
                r = 2 * level + d
                if p == 0:
                    acc[r, pl.ds(s16, 16)] = v
                else:
                    plsc.addupdate(acc.at[r, pl.ds(s16, 16)], v)


def _body(xt, tabf, out, xbuf, frac0, frac1, idxe0, idxo0, idxe1, idxo1,
          rows00, rows01, rows10, rows11, acc, sem00, sem01, sem10, sem11):
    wid = lax.axis_index("s") * _NC + lax.axis_index("c")
    fracs = (frac0, frac1)
    idxes, idxos = (idxe0, idxe1), (idxo0, idxo1)
    rowss = ((rows00, rows01), (rows10, rows11))
    sems = ((sem00, sem01), (sem10, sem11))

    @pl.loop(0, _NCHUNKS)
    def _chunk(k):
        base = wid * _PTS_PER_W + k * _CHUNK
        pltpu.sync_copy(xt.at[:, pl.ds(base, _CHUNK)], xbuf)

        @pl.loop(0, _NGROUPS)
        def _clamp(i):
            for c in range(3):
                v = xbuf[c, pl.ds(i * 16, 16)]
                xbuf[c, pl.ds(i * 16, 16)] = jnp.minimum(
                    jnp.maximum(v, 0.0), 1.0)

        prev = None
        for l in range(_NUM_LEVELS):
            slot = l % 2
            _phase_a(l, xbuf, fracs[slot], idxes[slot], idxos[slot])
            cpe = pltpu.make_async_copy(tabf.at[idxes[slot]],
                                        rowss[slot][0], sems[slot][0])
            cpo = pltpu.make_async_copy(tabf.at[idxos[slot]],
                                        rowss[slot][1], sems[slot][1])
            cpe.start()
            cpo.start()
            if prev is not None:
                pcpe, pcpo, pl_, pslot = prev
                pcpe.wait()
                pcpo.wait()
                _phase_b(pl_, fracs[pslot], rowss[pslot][0], rowss[pslot][1],
                         acc)
            prev = (cpe, cpo, l, slot)
        pcpe, pcpo, pl_, pslot = prev
        pcpe.wait()
        pcpo.wait()
        _phase_b(pl_, fracs[pslot], rowss[pslot][0], rowss[pslot][1], acc)

        pltpu.sync_copy(acc, out.at[:, pl.ds(base, _CHUNK)])


@jax.jit
def kernel(x, tables):
    n_planes, rows, ld = tables.shape
    xt = x.T
    tabf = tables.reshape(n_planes * rows * ld)
    mesh = plsc.VectorSubcoreMesh(core_axis_name="c", subcore_axis_name="s",
                                  num_cores=_NC, num_subcores=_NS)
    run = pl.kernel(
        _body,
        out_type=jax.ShapeDtypeStruct((2 * _NUM_LEVELS, _B), jnp.float32),
        mesh=mesh,
        scratch_types=[
            pltpu.VMEM((3, _CHUNK), jnp.float32),
            pltpu.VMEM((3, _CHUNK), jnp.float32),
            pltpu.VMEM((3, _CHUNK), jnp.float32),
            pltpu.VMEM((_NIDX,), jnp.int32),
            pltpu.VMEM((_NIDX,), jnp.int32),
            pltpu.VMEM((_NIDX,), jnp.int32),
            pltpu.VMEM((_NIDX,), jnp.int32),
            pltpu.VMEM((_NIDX,), jnp.float32),
            pltpu.VMEM((_NIDX,), jnp.float32),
            pltpu.VMEM((_NIDX,), jnp.float32),
            pltpu.VMEM((_NIDX,), jnp.float32),
            pltpu.VMEM((2 * _NUM_LEVELS, _CHUNK), jnp.float32),
            pltpu.SemaphoreType.DMA,
            pltpu.SemaphoreType.DMA,
            pltpu.SemaphoreType.DMA,
            pltpu.SemaphoreType.DMA,
        ],
    )
    out_t = run(xt, tabf)
    return out_t.T


# speedup vs baseline: 1.0335x; 1.0335x over previous
import math

import jax
import jax.numpy as jnp
from jax import lax
from jax.experimental import pallas as pl
from jax.experimental.pallas import tpu as pltpu
from jax.experimental.pallas import tpu_sc as plsc

_NUM_LEVELS = 16
_BASE_RES = 16
_DESIRED_RES = 2048
_LOG2_HASH = 19
_PRIME_I32 = 2654435761 - (1 << 32)
_HASH_MASK = (1 << _LOG2_HASH) - 1

_B = 262144
_NC, _NS = 2, 16
_NW = _NC * _NS
_CHUNK = 512
_PTS_PER_W = _B // _NW
_NCHUNKS = _PTS_PER_W // _CHUNK
_NGROUPS = _CHUNK // 16
_PLANES = ((0, 1), (0, 2), (1, 2))
_NIDX = 3 * 4 * _CHUNK


def _levels():
    L = _NUM_LEVELS
    b = math.exp(math.log(_DESIRED_RES / _BASE_RES) / (L - 1))
    H = 1 << _LOG2_HASH
    out, off = [], 0
    for l in range(L):
        res = int(math.ceil(_BASE_RES * (b ** l)))
        hashed = (res + 1) ** 2 > H
        size = H if hashed else (res + 1) ** 2
        out.append((res, hashed, off))
        off += size
    return out, off


_LEVEL_PARAMS, _ROWS_PER_PLANE = _levels()


def _phase_a(level, xbuf, fracb, idxe, idxo):
    res, hashed, loff = _LEVEL_PARAMS[level]
    resf = float(res)
    stride = res + 1

    @pl.loop(0, _NGROUPS)
    def _g(i):
        s16 = i * 16
        ip0, ip1 = [], []
        for c in range(3):
            xv = xbuf[pl.ds(c * _CHUNK + s16, 16)]
            pos = xv * resf
            ip = pos.astype(jnp.int32)
            fracb[c, pl.ds(s16, 16)] = pos - ip.astype(jnp.float32)
            ip0.append(ip)
            ip1.append(jnp.minimum(ip + 1, res))
        for p, (ca, cb) in enumerate(_PLANES):
            off = p * _ROWS_PER_PLANE + loff
            a0, a1 = ip0[ca], ip1[ca]
            if not hashed:
                b0 = ip0[cb] * stride + off
                b1 = ip1[cb] * stride + off
                corners = (a0 + b0, a0 + b1, a1 + b0, a1 + b1)
            else:
                h0 = ip0[cb] * _PRIME_I32
                h1 = ip1[cb] * _PRIME_I32
                corners = (((a0 ^ h0) & _HASH_MASK) + off,
                           ((a0 ^ h1) & _HASH_MASK) + off,
                           ((a1 ^ h0) & _HASH_MASK) + off,
                           ((a1 ^ h1) & _HASH_MASK) + off)
            for c4, v in enumerate(corners):
                e = v + v
                idxe[pl.ds(p * 4 * _CHUNK + c4 * _CHUNK + s16, 16)] = e
                idxo[pl.ds(p * 4 * _CHUNK + c4 * _CHUNK + s16, 16)] = e + 1


def _phase_b(level, fracb, rows0, rows1, accf):
    @pl.loop(0, _NGROUPS)
    def _g(i):
        s16 = i * 16
        for d, rows in enumerate((rows0, rows1)):
            r = 2 * level + d
            acc = None
            for p, (ca, cb) in enumerate(_PLANES):
                fx = fracb[ca, pl.ds(s16, 16)]
                fy = fracb[cb, pl.ds(s16, 16)]
                base = p * 4 * _CHUNK + s16
                g00 = rows[pl.ds(base, 16)]
                g01 = rows[pl.ds(base + _CHUNK, 16)]
                g10 = rows[pl.ds(base + 2 * _CHUNK, 16)]
                g11 = rows[pl.ds(base + 3 * _CHUNK, 16)]
                v0 = g00 + fy * (g01 - g00)
                v1 = g10 + fy * (g11 - g10)
                v = v0 + fx * (v1 - v0)
                acc = v if acc is None else acc + v
            accf[r, pl.ds(s16, 16)] = acc


def _body(xf, tabf, out, xidx, xbuf, frac0, frac1, idxe0, idxo0, idxe1,
          idxo1, rows00, rows01, rows10, rows11, accf, semx, sem00, sem01,
          sem10, sem11):
    wid = lax.axis_index("s") * _NC + lax.axis_index("c")
    fracs = (frac0, frac1)
    idxes, idxos = (idxe0, idxe1), (idxo0, idxo1)
    rowss = ((rows00, rows01), (rows10, rows11))
    sems = ((sem00, sem01), (sem10, sem11))

    @pl.loop(0, _NCHUNKS)
    def _chunk(k):
        base = wid * _PTS_PER_W + k * _CHUNK

        @pl.loop(0, _NGROUPS)
        def _xi(i):
            j3 = (lax.iota(jnp.int32, 16) + (base + i * 16)) * 3
            for c in range(3):
                xidx[pl.ds(c * _CHUNK + i * 16, 16)] = j3 + c

        cpx = pltpu.make_async_copy(xf.at[xidx], xbuf, semx)
        cpx.start()
        cpx.wait()

        @pl.loop(0, 3 * _NGROUPS)
        def _clamp(i):
            v = xbuf[pl.ds(i * 16, 16)]
            xbuf[pl.ds(i * 16, 16)] = jnp.minimum(jnp.maximum(v, 0.0), 1.0)

        prev = None
        for l in range(_NUM_LEVELS):
            slot = l % 2
            _phase_a(l, xbuf, fracs[slot], idxes[slot], idxos[slot])
            cpe = pltpu.make_async_copy(tabf.at[idxes[slot]],
                                        rowss[slot][0], sems[slot][0])
            cpo = pltpu.make_async_copy(tabf.at[idxos[slot]],
                                        rowss[slot][1], sems[slot][1])
            cpe.start()
            cpo.start()
            if prev is not None:
                pcpe, pcpo, pl_, pslot = prev
                pcpe.wait()
                pcpo.wait()
                _phase_b(pl_, fracs[pslot], rowss[pslot][0], rowss[pslot][1],
                         accf)
            prev = (cpe, cpo, l, slot)
        pcpe, pcpo, pl_, pslot = prev
        pcpe.wait()
        pcpo.wait()
        _phase_b(pl_, fracs[pslot], rowss[pslot][0], rowss[pslot][1], accf)

        pltpu.sync_copy(accf, out.at[:, pl.ds(base, _CHUNK)])


@jax.jit
def kernel(x, tables):
    n_planes, rows, ld = tables.shape
    xf = x.reshape(_B * 3)
    tabf = tables.reshape(n_planes * rows * ld)
    mesh = plsc.VectorSubcoreMesh(core_axis_name="c", subcore_axis_name="s",
                                  num_cores=_NC, num_subcores=_NS)
    run = pl.kernel(
        _body,
        out_type=jax.ShapeDtypeStruct((2 * _NUM_LEVELS, _B), jnp.float32),
        mesh=mesh,
        scratch_types=[
            pltpu.VMEM((3 * _CHUNK,), jnp.int32),
            pltpu.VMEM((3 * _CHUNK,), jnp.float32),
            pltpu.VMEM((3, _CHUNK), jnp.float32),
            pltpu.VMEM((3, _CHUNK), jnp.float32),
            pltpu.VMEM((_NIDX,), jnp.int32),
            pltpu.VMEM((_NIDX,), jnp.int32),
            pltpu.VMEM((_NIDX,), jnp.int32),
            pltpu.VMEM((_NIDX,), jnp.int32),
            pltpu.VMEM((_NIDX,), jnp.float32),
            pltpu.VMEM((_NIDX,), jnp.float32),
            pltpu.VMEM((_NIDX,), jnp.float32),
            pltpu.VMEM((_NIDX,), jnp.float32),
            pltpu.VMEM((2 * _NUM_LEVELS, _CHUNK), jnp.float32),
            pltpu.SemaphoreType.DMA,
            pltpu.SemaphoreType.DMA,
            pltpu.SemaphoreType.DMA,
            pltpu.SemaphoreType.DMA,
            pltpu.SemaphoreType.DMA,
        ],
    )
    out_f = run(xf, tabf)
    return out_f.T
